# Initial kernel scaffold; baseline (speedup 1.0000x reference)
import jax
import jax.numpy as jnp
from jax.experimental import pallas as pl


def kernel(h_bond, bond_index, h_node, bond_time, L_Wb, L_Wn, L_W1, L_b1, L_W2, L_b2, L_Wg1, L_bg1, L_Wg2, L_bg2, R_Wb, R_Wn, R_W1, R_b1, R_W2, R_b2, R_Wg1, R_bg1, R_Wg2, R_bg2, nl_W, nl_b, nr_W, nr_b, sf_W, sf_b, ln_g, ln_b, ot_W, ot_b):
    raise NotImplementedError("write your pallas kernel here")


# speedup vs baseline: 1.0807x; 1.0807x over previous
import jax
import jax.numpy as jnp
from jax.experimental import pallas as pl
from jax.experimental.pallas import tpu as pltpu

E = 160000
N = 10000
D_EDGE = 128
D_NODE = 128
D_INTER = 256

BE = 1000


def _sigmoid(x):
    return 1.0 / (1.0 + jnp.exp(-x))


def _tc1_body(hb_ref, hl_ref, hr_ref, tm_ref,
              LWb, LWn, LW1, Lb1, LW2, Lb2, LG1b, LG1n, LG1t, Lbg1, LG2, Lbg2,
              RWb, RWn, RW1, Rb1, RW2, Rb2, RG1b, RG1n, RG1t, Rbg1, RG2, Rbg2,
              nlW, nlb, nrW, nrb, sfW, sfb,
              msgl_out, msgr_out, hd_out):
    hb = hb_ref[...]
    hl = hl_ref[...]
    hr = hr_ref[...]
    tm = tm_ref[...]

    def side(hn, Wb, Wn, W1, b1, W2, b2, G1b, G1n, G1t, bg1, G2, bg2):
        inter = (jnp.dot(hb, Wb[...], preferred_element_type=jnp.float32)
                 * jnp.dot(hn, Wn[...], preferred_element_type=jnp.float32))
        inter = jnp.maximum(
            jnp.dot(inter, W1[...], preferred_element_type=jnp.float32) + b1[...], 0.0)
        inter = jnp.dot(inter, W2[...], preferred_element_type=jnp.float32) + b2[...]
        g = (jnp.dot(hb, G1b[...], preferred_element_type=jnp.float32)
             + jnp.dot(hn, G1n[...], preferred_element_type=jnp.float32)
             + tm * G1t[...] + bg1[...])
        g = jnp.maximum(g, 0.0)
        g = jnp.dot(g, G2[...], preferred_element_type=jnp.float32) + bg2[...]
        return inter * _sigmoid(g)

    msgl_out[...] = side(hl, LWb, LWn, LW1, Lb1, LW2, Lb2, LG1b, LG1n, LG1t,
                         Lbg1, LG2, Lbg2)
    msgr_out[...] = side(hr, RWb, RWn, RW1, Rb1, RW2, Rb2, RG1b, RG1n, RG1t,
                         Rbg1, RG2, Rbg2)
    hd_out[...] = (jnp.dot(hl, nlW[...], preferred_element_type=jnp.float32) + nlb[...]
                   + jnp.dot(hr, nrW[...], preferred_element_type=jnp.float32) + nrb[...]
                   + jnp.dot(hb, sfW[...], preferred_element_type=jnp.float32) + sfb[...])


def _tc2_body(ml_ref, mr_ref, hd_ref, ln_g, ln_b, ot_W, ot_b, out_ref):
    h = ml_ref[...] + mr_ref[...] + hd_ref[...]
    mu = jnp.mean(h, axis=-1, keepdims=True)
    var = jnp.mean((h - mu) ** 2, axis=-1, keepdims=True)
    h = (h - mu) * jax.lax.rsqrt(var + 1e-5) * ln_g[...] + ln_b[...]
    h = jnp.maximum(h, 0.0)
    out_ref[...] = jnp.dot(h, ot_W[...], preferred_element_type=jnp.float32) + ot_b[...]


def _edge_spec(d):
    return pl.BlockSpec((BE, d), lambda i: (i, 0))


def _full_spec(shape):
    nd = len(shape)
    return pl.BlockSpec(shape, lambda i: (0,) * nd)


def kernel(h_bond, bond_index, h_node, bond_time,
           L_Wb, L_Wn, L_W1, L_b1, L_W2, L_b2, L_Wg1, L_bg1, L_Wg2, L_bg2,
           R_Wb, R_Wn, R_W1, R_b1, R_W2, R_b2, R_Wg1, R_bg1, R_Wg2, R_bg2,
           nl_W, nl_b, nr_W, nr_b, sf_W, sf_b, ln_g, ln_b, ot_W, ot_b):
    left = bond_index[0]
    right = bond_index[1]

    LG1b, LG1n, LG1t = L_Wg1[:D_EDGE], L_Wg1[D_EDGE:D_EDGE + D_NODE], L_Wg1[D_EDGE + D_NODE:]
    RG1b, RG1n, RG1t = R_Wg1[:D_EDGE], R_Wg1[D_EDGE:D_EDGE + D_NODE], R_Wg1[D_EDGE + D_NODE:]

    def row(b):
        return b.reshape(1, -1)

    hl = h_node[left]
    hr = h_node[right]

    grid = E // BE
    msgl, msgr, hdense = pl.pallas_call(
        _tc1_body,
        grid=(grid,),
        in_specs=[
            _edge_spec(D_EDGE), _edge_spec(D_NODE), _edge_spec(D_NODE), _edge_spec(1),
            _full_spec((D_EDGE, D_INTER)), _full_spec((D_NODE, D_INTER)),
            _full_spec((D_INTER, D_INTER)), _full_spec((1, D_INTER)),
            _full_spec((D_INTER, D_EDGE)), _full_spec((1, D_EDGE)),
            _full_spec((D_EDGE, 32)), _full_spec((D_NODE, 32)),
            _full_spec((1, 32)), _full_spec((1, 32)),
            _full_spec((32, D_EDGE)), _full_spec((1, D_EDGE)),
            _full_spec((D_EDGE, D_INTER)), _full_spec((D_NODE, D_INTER)),
            _full_spec((D_INTER, D_INTER)), _full_spec((1, D_INTER)),
            _full_spec((D_INTER, D_EDGE)), _full_spec((1, D_EDGE)),
            _full_spec((D_EDGE, 32)), _full_spec((D_NODE, 32)),
            _full_spec((1, 32)), _full_spec((1, 32)),
            _full_spec((32, D_EDGE)), _full_spec((1, D_EDGE)),
            _full_spec((D_NODE, D_EDGE)), _full_spec((1, D_EDGE)),
            _full_spec((D_NODE, D_EDGE)), _full_spec((1, D_EDGE)),
            _full_spec((D_EDGE, D_EDGE)), _full_spec((1, D_EDGE)),
        ],
        out_specs=[_edge_spec(D_EDGE), _edge_spec(D_EDGE), _edge_spec(D_EDGE)],
        out_shape=[jax.ShapeDtypeStruct((E, D_EDGE), jnp.float32)] * 3,
    )(h_bond, hl, hr, bond_time,
      L_Wb, L_Wn, L_W1, row(L_b1), L_W2, row(L_b2),
      LG1b, LG1n, LG1t, row(L_bg1), L_Wg2, row(L_bg2),
      R_Wb, R_Wn, R_W1, row(R_b1), R_W2, row(R_b2),
      RG1b, RG1n, RG1t, row(R_bg1), R_Wg2, row(R_bg2),
      nl_W, row(nl_b), nr_W, row(nr_b), sf_W, row(sf_b))

    agg_l = jax.ops.segment_sum(msgl, right, num_segments=N)
    agg_r = jax.ops.segment_sum(msgr, left, num_segments=N)
    msg_l = agg_l[left]
    msg_r = agg_r[right]

    out = pl.pallas_call(
        _tc2_body,
        grid=(grid,),
        in_specs=[
            _edge_spec(D_EDGE), _edge_spec(D_EDGE), _edge_spec(D_EDGE),
            _full_spec((1, D_EDGE)), _full_spec((1, D_EDGE)),
            _full_spec((D_EDGE, D_EDGE)), _full_spec((1, D_EDGE)),
        ],
        out_specs=_edge_spec(D_EDGE),
        out_shape=jax.ShapeDtypeStruct((E, D_EDGE), jnp.float32),
    )(msg_l, msg_r, hdense, row(ln_g), row(ln_b), ot_W, row(ot_b))
    return out
